# baseline (speedup 1.0000x reference)
import functools

import jax
import jax.numpy as jnp
from jax import lax
from jax.experimental import pallas as pl
from jax.experimental.pallas import tpu as pltpu
from jax.experimental.pallas import tpu_sc as plsc

_B = 4096
_C = 1000
_BB = 1024
_GRID = _B // _BB

_NC = 2
_NS = 16
_NW = _NC * _NS
_RG = _C // 8
_GPW = -(-_RG // _NW)


def _fused_body(xt_ref, lab_ref, coarse_ref, fine_ref, svt_ref, loss_ref, fb_ref):
    i = pl.program_id(0)

    @pl.when(i == 0)
    def _():
        fb_ref[...] = fine_ref[...].astype(jnp.bfloat16)
        loss_ref[0, 0] = 0.0

    xt = xt_ref[...]
    idx = jnp.argmax(xt, axis=0).astype(jnp.int32)
    classes = jax.lax.broadcasted_iota(jnp.int32, (_C, _BB), 0)
    onehot = jnp.where(
        classes == idx[None, :], coarse_ref[...], 0.0
    ).astype(jnp.bfloat16)
    denom = jax.lax.dot_general(
        fb_ref[...], onehot, (((0,), (0,)), ((), ())),
        preferred_element_type=jnp.float32,
    )
    svt = xt / denom
    svt_ref[...] = svt

    lab = lab_ref[0, 0, :].astype(jnp.int32)
    sel = jnp.sum(jnp.where(lab[None, :] == classes, svt, 0.0), axis=0)
    col_max = jnp.max(svt, axis=0)
    lse = col_max + jnp.log(jnp.sum(jnp.exp(svt - col_max[None, :]), axis=0))
    loss_ref[0, 0] += jnp.sum(lse - sel) / _B


def _tc_pass(xt, labels3, coarse2, fine):
    return pl.pallas_call(
        _fused_body,
        grid=(_GRID,),
        in_specs=[
            pl.BlockSpec((_C, _BB), lambda i: (0, i)),
            pl.BlockSpec((1, 1, _BB), lambda i: (i, 0, 0)),
            pl.BlockSpec((_C, 1), lambda i: (0, 0)),
            pl.BlockSpec((_C, _C), lambda i: (0, 0)),
        ],
        out_specs=[
            pl.BlockSpec((_C, _BB), lambda i: (0, i)),
            pl.BlockSpec(memory_space=pltpu.SMEM),
        ],
        out_shape=[
            jax.ShapeDtypeStruct((_C, _B), jnp.float32),
            jax.ShapeDtypeStruct((1, 1), jnp.float32),
        ],
        scratch_shapes=[pltpu.VMEM((_C, _C), jnp.bfloat16)],
    )(xt, labels3, coarse2, fine)


def _sc_reg_partials(fine):
    mesh = plsc.VectorSubcoreMesh(core_axis_name="c", subcore_axis_name="s")

    @functools.partial(
        pl.kernel,
        mesh=mesh,
        out_type=jax.ShapeDtypeStruct((_NW, 16), jnp.float32),
        scratch_types=[
            pltpu.VMEM((8, _C), jnp.float32),
            pltpu.VMEM((16,), jnp.float32),
        ],
    )
    def k(fine_hbm, out_hbm, buf, accv):
        wid = lax.axis_index("s") * _NC + lax.axis_index("c")
        lane = lax.iota(jnp.int32, 16)
        accv[...] = jnp.zeros((16,), jnp.float32)
        for g in range(_GPW):
            grp = wid + _NW * g

            @pl.when(grp < _RG)
            def _():
                pltpu.sync_copy(fine_hbm.at[pl.ds(grp * 8, 8)], buf)
                for r in range(8):
                    def body(j, c):
                        accv[...] += jnp.abs(buf[r, pl.ds(j * 16, 16)] - 1.0)
                        return c
                    lax.fori_loop(0, (_C // 16), body, 0)
                    tail = jnp.abs(buf[r, pl.ds(_C - 16, 16)] - 1.0)
                    accv[...] += jnp.where(lane >= 16 - _C % 16, tail, 0.0)
        pltpu.sync_copy(accv, out_hbm.at[wid])

    return k(fine)


def kernel(Simple_vector, label_list, coarse_scaling_vector, fine_scaling_matrix):
    labels3 = label_list.reshape(_GRID, 1, _BB)
    svt, nll = _tc_pass(
        Simple_vector.T,
        labels3,
        coarse_scaling_vector[:, None],
        fine_scaling_matrix,
    )
    reg_parts = _sc_reg_partials(fine_scaling_matrix)
    loss = nll.reshape(()) + reg_parts.sum() / (_C * _C)
    softmaxed = jnp.zeros((), dtype=svt.dtype)
    return (svt.T, loss, softmaxed)


# speedup vs baseline: 1.7849x; 1.7849x over previous
import jax
import jax.numpy as jnp
from jax.experimental import pallas as pl
from jax.experimental.pallas import tpu as pltpu

_B = 4096
_C = 1000
_BB = 1024
_GRID = _B // _BB


def _fused_body(xt_ref, lab_ref, coarse_ref, fine_ref, svt_ref, loss_ref, fb_ref):
    i = pl.program_id(0)

    @pl.when(i == 0)
    def _():
        fb_ref[...] = fine_ref[...].astype(jnp.bfloat16)

    xt = xt_ref[...]
    idx = jnp.argmax(xt, axis=0).astype(jnp.int32)
    classes = jax.lax.broadcasted_iota(jnp.int32, (_C, _BB), 0)
    onehot = jnp.where(
        classes == idx[None, :], coarse_ref[...], 0.0
    ).astype(jnp.bfloat16)
    denom = jax.lax.dot_general(
        fb_ref[...], onehot, (((0,), (0,)), ((), ())),
        preferred_element_type=jnp.float32,
    )
    svt = xt / denom
    svt_ref[...] = svt

    lab = lab_ref[0, 0, :].astype(jnp.int32)
    sel = jnp.sum(jnp.where(lab[None, :] == classes, svt, 0.0), axis=0)
    col_max = jnp.max(svt, axis=0)
    lse = col_max + jnp.log(jnp.sum(jnp.exp(svt - col_max[None, :]), axis=0))
    part = jnp.sum(lse - sel)

    @pl.when(i == 0)
    def _():
        reg = jnp.sum(jnp.abs(fine_ref[...] - 1.0))
        loss_ref[0, 0] = reg / (_C * _C)

    loss_ref[0, 0] += part / _B


def kernel(Simple_vector, label_list, coarse_scaling_vector, fine_scaling_matrix):
    labels3 = label_list.reshape(_GRID, 1, _BB)
    svt, loss = pl.pallas_call(
        _fused_body,
        grid=(_GRID,),
        in_specs=[
            pl.BlockSpec((_C, _BB), lambda i: (0, i)),
            pl.BlockSpec((1, 1, _BB), lambda i: (i, 0, 0)),
            pl.BlockSpec((_C, 1), lambda i: (0, 0)),
            pl.BlockSpec((_C, _C), lambda i: (0, 0)),
        ],
        out_specs=[
            pl.BlockSpec((_C, _BB), lambda i: (0, i)),
            pl.BlockSpec(memory_space=pltpu.SMEM),
        ],
        out_shape=[
            jax.ShapeDtypeStruct((_C, _B), jnp.float32),
            jax.ShapeDtypeStruct((1, 1), jnp.float32),
        ],
        scratch_shapes=[pltpu.VMEM((_C, _C), jnp.bfloat16)],
    )(
        Simple_vector.T,
        labels3,
        coarse_scaling_vector[:, None],
        fine_scaling_matrix,
    )
    softmaxed = jnp.zeros((), dtype=svt.dtype)
    return (svt.T, loss.reshape(()), softmaxed)


# speedup vs baseline: 1.8082x; 1.0131x over previous
import jax
import jax.numpy as jnp
from jax.experimental import pallas as pl
from jax.experimental.pallas import tpu as pltpu

_B = 4096
_C = 1000
_BB = 1024
_GRID = _B // _BB


def _fused_body(xt_ref, lab_ref, coarse_ref, fine_ref, svt_ref, loss_ref, fb_ref):
    i = pl.program_id(0)

    @pl.when(i == 0)
    def _():
        fb_ref[...] = (1.0 / (coarse_ref[...] * fine_ref[...])).astype(jnp.bfloat16)

    xt = xt_ref[...]
    idx = jnp.argmax(xt, axis=0).astype(jnp.int32)
    classes = jax.lax.broadcasted_iota(jnp.int32, (_C, _BB), 0)
    onehot = jnp.where(
        classes == idx[None, :], 1.0, 0.0
    ).astype(jnp.bfloat16)
    rdenom = jax.lax.dot_general(
        fb_ref[...], onehot, (((0,), (0,)), ((), ())),
        preferred_element_type=jnp.float32,
    )
    svt = xt * rdenom
    svt_ref[...] = svt

    lab = lab_ref[0, 0, :].astype(jnp.int32)
    sel = jnp.sum(jnp.where(lab[None, :] == classes, svt, 0.0), axis=0)
    col_max = jnp.max(svt, axis=0)
    lse = col_max + jnp.log(jnp.sum(jnp.exp(svt - col_max[None, :]), axis=0))
    part = jnp.sum(lse - sel)

    @pl.when(i == 0)
    def _():
        reg = jnp.sum(jnp.abs(fine_ref[...] - 1.0))
        loss_ref[0, 0] = reg / (_C * _C)

    loss_ref[0, 0] += part / _B


def kernel(Simple_vector, label_list, coarse_scaling_vector, fine_scaling_matrix):
    labels3 = label_list.reshape(_GRID, 1, _BB)
    svt, loss = pl.pallas_call(
        _fused_body,
        grid=(_GRID,),
        in_specs=[
            pl.BlockSpec((_C, _BB), lambda i: (0, i)),
            pl.BlockSpec((1, 1, _BB), lambda i: (i, 0, 0)),
            pl.BlockSpec((_C, 1), lambda i: (0, 0)),
            pl.BlockSpec((_C, _C), lambda i: (0, 0)),
        ],
        out_specs=[
            pl.BlockSpec((_C, _BB), lambda i: (0, i)),
            pl.BlockSpec(memory_space=pltpu.SMEM),
        ],
        out_shape=[
            jax.ShapeDtypeStruct((_C, _B), jnp.float32),
            jax.ShapeDtypeStruct((1, 1), jnp.float32),
        ],
        scratch_shapes=[pltpu.VMEM((_C, _C), jnp.bfloat16)],
    )(
        Simple_vector.T,
        labels3,
        coarse_scaling_vector[:, None],
        fine_scaling_matrix,
    )
    softmaxed = jnp.zeros((), dtype=svt.dtype)
    return (svt.T, loss.reshape(()), softmaxed)
